# baseline (speedup 1.0000x reference)
import functools

import jax
import jax.numpy as jnp
from jax import lax
from jax.experimental import pallas as pl
from jax.experimental.pallas import tpu as pltpu
from jax.experimental.pallas import tpu_sc as plsc

D = 64
LANES = 16
NC = 2
NS = 16
NW = NC * NS

S = 128
BM = 1024
HP = 500736


def _repack_body(sl_ref, sr_ref, a_ref, b_ref, o_ref):
    sl = jnp.maximum(sl_ref[...], jnp.float32(1e-8))
    sr = jnp.maximum(sr_ref[...], jnp.float32(1e-8))
    a = a_ref[...].T
    b = b_ref[...].T
    lq = jnp.where(a < 0, -sl, sl)
    rq = jnp.where(b < 0, -sr, sr)
    o_ref[...] = jnp.concatenate([lq, rq], axis=1)


@functools.lru_cache(maxsize=None)
def _make_repack(V):
    n_blk = HP // BM
    last = (V - 1) // BM
    return pl.pallas_call(
        _repack_body,
        grid=(n_blk,),
        in_specs=[
            pl.BlockSpec((BM, 1), lambda g: (g, 0)),
            pl.BlockSpec((BM, 1), lambda g: (jnp.minimum(g + n_blk, last), 0)),
            pl.BlockSpec((D, BM), lambda g: (0, g)),
            pl.BlockSpec((D, BM), lambda g: (0, jnp.minimum(g + n_blk, last))),
        ],
        out_specs=pl.BlockSpec((BM, 2 * D), lambda g: (g, 0)),
        out_shape=jax.ShapeDtypeStruct((HP, 2 * D), jnp.float32),
    )


@functools.lru_cache(maxsize=None)
def _make_gather(N):
    assert N % (NW * S) == 0
    per_w = N // NW
    n_chunks = per_w // S
    mesh = plsc.VectorSubcoreMesh(core_axis_name="c", subcore_axis_name="s")

    @functools.partial(
        pl.kernel,
        mesh=mesh,
        compiler_params=pltpu.CompilerParams(use_tc_tiling_on_sc=True),
        out_type=jax.ShapeDtypeStruct((N, D), jnp.float32),
        scratch_types=[
            pltpu.VMEM((per_w,), jnp.int32),
            pltpu.VMEM((S,), jnp.int32),
            pltpu.VMEM((S, 2 * D), jnp.float32),
            pltpu.VMEM((S, D), jnp.float32),
            pltpu.SemaphoreType.DMA,
        ],
    )
    def k(ids_hbm, w_hbm, out_hbm, idx_v, hidx_v, rows_v, outb_v, sem):
        wid = lax.axis_index("s") * NC + lax.axis_index("c")
        base = wid * per_w

        pltpu.sync_copy(ids_hbm.at[pl.ds(base, per_w)], idx_v)

        def chunk_body(c, carry):
            def half_body(i, carry2):
                ig = idx_v[pl.ds(c * S + i * LANES, LANES)]
                wrap = jnp.where(ig >= HP, HP, 0)
                hidx_v[pl.ds(i * LANES, LANES)] = ig - wrap
                return carry2

            lax.fori_loop(0, S // LANES, half_body, 0, unroll=2)

            pltpu.async_copy(w_hbm.at[hidx_v], rows_v, sem).wait()

            def group_body(g, carry2):
                ig = idx_v[pl.ds(c * S + g * LANES, LANES)]
                hg = jnp.where(ig >= HP, D, 0)
                for kk in range(LANES):
                    r = g * LANES + kk
                    h = hg[kk]
                    for j in range(D // LANES):
                        outb_v[r, pl.ds(j * LANES, LANES)] = (
                            rows_v[r, pl.ds(h + j * LANES, LANES)])
                return carry2

            lax.fori_loop(0, S // LANES, group_body, 0)

            pltpu.sync_copy(outb_v, out_hbm.at[pl.ds(base + c * S, S)])
            return carry

        lax.fori_loop(0, n_chunks, chunk_body, 0)

    return k


def kernel(input_ids, weight, scales):
    B, L = input_ids.shape
    V = weight.shape[0]
    N = B * L
    ids = input_ids.reshape(N).astype(jnp.int32)
    wt = weight.T
    w2 = _make_repack(V)(scales, scales, wt, wt)
    out = _make_gather(N)(ids, w2)
    return out.reshape(B, L, D)


# speedup vs baseline: 1.4013x; 1.4013x over previous
import functools

import jax
import jax.numpy as jnp
from jax import lax
from jax.experimental import pallas as pl
from jax.experimental.pallas import tpu as pltpu
from jax.experimental.pallas import tpu_sc as plsc

D = 64
LANES = 16
NC = 2
NS = 16
NW = NC * NS

S = 128


@functools.lru_cache(maxsize=None)
def _make_gather(N, V):
    assert N % (NW * S) == 0
    per_w = N // NW
    n_chunks = per_w // S
    mesh = plsc.VectorSubcoreMesh(core_axis_name="c", subcore_axis_name="s")

    @functools.partial(
        pl.kernel,
        mesh=mesh,
        compiler_params=pltpu.CompilerParams(use_tc_tiling_on_sc=True),
        out_type=jax.ShapeDtypeStruct((N, D), jnp.float32),
        scratch_types=[
            pltpu.VMEM((per_w,), jnp.int32),
            pltpu.VMEM((S, 2 * D), jnp.float32),
            pltpu.VMEM((S, D), jnp.float32),
            pltpu.VMEM((S,), jnp.float32),
            pltpu.SemaphoreType.DMA,
        ],
    )
    def k(ids_hbm, w_hbm, sc_hbm, out_hbm,
          idx_v, rows_v, outb_v, scf_v, sem):
        wid = lax.axis_index("s") * NC + lax.axis_index("c")
        base = wid * per_w

        pltpu.sync_copy(ids_hbm.at[pl.ds(base, per_w)], idx_v)

        def chunk_body(c, carry):
            idx_c = idx_v.at[pl.ds(c * S, S)]

            pltpu.async_copy(sc_hbm.at[idx_c], scf_v, sem).wait()
            pltpu.async_copy(w_hbm.at[idx_c], rows_v, sem).wait()

            def group_body(g, carry2):
                sg = jnp.maximum(scf_v[pl.ds(g * LANES, LANES)],
                                 jnp.float32(1e-8))
                nsg = -sg
                for kk in range(LANES):
                    splat = jnp.broadcast_to(sg[kk], (LANES,))
                    nsplat = jnp.broadcast_to(nsg[kk], (LANES,))
                    r = g * LANES + kk
                    for j in range(D // LANES):
                        w = rows_v[r, pl.ds(j * LANES, LANES)]
                        outb_v[r, pl.ds(j * LANES, LANES)] = jnp.where(
                            w < 0, nsplat, splat)
                return carry2

            lax.fori_loop(0, S // LANES, group_body, 0)

            pltpu.sync_copy(outb_v, out_hbm.at[pl.ds(base + c * S, S)])
            return carry

        lax.fori_loop(0, n_chunks, chunk_body, 0)

    return k


def kernel(input_ids, weight, scales):
    B, L = input_ids.shape
    V = weight.shape[0]
    N = B * L
    ids = input_ids.reshape(N).astype(jnp.int32)
    sc_flat = scales.reshape(-1)
    w128 = jnp.pad(weight, ((0, 0), (0, 2 * D - weight.shape[1])))
    out = _make_gather(N, V)(ids, w128, sc_flat)
    return out.reshape(B, L, D)


# speedup vs baseline: 1.5911x; 1.1354x over previous
import functools

import jax
import jax.numpy as jnp
from jax import lax
from jax.experimental import pallas as pl
from jax.experimental.pallas import tpu as pltpu
from jax.experimental.pallas import tpu_sc as plsc

D = 64
LANES = 16
NC = 2
NS = 16
NW = NC * NS

S = 128


@functools.lru_cache(maxsize=None)
def _make_gather(N, V):
    assert N % (NW * S) == 0
    per_w = N // NW
    n_chunks = per_w // S
    mesh = plsc.VectorSubcoreMesh(core_axis_name="c", subcore_axis_name="s")

    @functools.partial(
        pl.kernel,
        mesh=mesh,
        compiler_params=pltpu.CompilerParams(use_tc_tiling_on_sc=True),
        out_type=jax.ShapeDtypeStruct((N, D), jnp.float32),
        scratch_types=[
            pltpu.VMEM((per_w,), jnp.int32),
            pltpu.VMEM((2, S, 2 * D), jnp.float32),
            pltpu.VMEM((S, D), jnp.float32),
            pltpu.VMEM((2, S), jnp.float32),
            pltpu.SemaphoreType.DMA,
            pltpu.SemaphoreType.DMA,
        ],
    )
    def k(ids_hbm, w_hbm, sc_hbm, out_hbm,
          idx_v, rows_v, outb_v, scf_v, sem0, sem1):
        wid = lax.axis_index("s") * NC + lax.axis_index("c")
        base = wid * per_w
        sems = (sem0, sem1)

        pltpu.sync_copy(ids_hbm.at[pl.ds(base, per_w)], idx_v)

        def fire(c, b):
            idx_c = idx_v.at[pl.ds(c * S, S)]
            pltpu.async_copy(sc_hbm.at[idx_c], scf_v.at[b], sems[b])
            pltpu.async_copy(w_hbm.at[idx_c], rows_v.at[b], sems[b])

        def drain(c, b):
            idx_c = idx_v.at[pl.ds(c * S, S)]
            pltpu.make_async_copy(sc_hbm.at[idx_c], scf_v.at[b],
                                  sems[b]).wait()
            pltpu.make_async_copy(w_hbm.at[idx_c], rows_v.at[b],
                                  sems[b]).wait()

        def compute(c, b):
            def group_body(g, carry2):
                sg = jnp.maximum(scf_v[b, pl.ds(g * LANES, LANES)],
                                 jnp.float32(1e-8))
                nsg = -sg
                for kk in range(LANES):
                    splat = jnp.broadcast_to(sg[kk], (LANES,))
                    nsplat = jnp.broadcast_to(nsg[kk], (LANES,))
                    r = g * LANES + kk
                    for j in range(D // LANES):
                        w = rows_v[b, r, pl.ds(j * LANES, LANES)]
                        outb_v[r, pl.ds(j * LANES, LANES)] = jnp.where(
                            w < 0, nsplat, splat)
                return carry2

            lax.fori_loop(0, S // LANES, group_body, 0)
            pltpu.sync_copy(outb_v, out_hbm.at[pl.ds(base + c * S, S)])

        fire(0, 0)

        def chunk_body(c2, carry):
            for b in range(2):
                c = c2 * 2 + b
                nxt = c + 1

                @pl.when(nxt < n_chunks)
                def _():
                    fire(nxt, 1 - b)

                drain(c, b)
                compute(c, b)
            return carry

        lax.fori_loop(0, n_chunks // 2, chunk_body, 0)

    return k


def kernel(input_ids, weight, scales):
    B, L = input_ids.shape
    V = weight.shape[0]
    N = B * L
    ids = input_ids.reshape(N).astype(jnp.int32)
    sc_flat = scales.reshape(-1)
    w128 = jnp.pad(weight, ((0, 0), (0, 2 * D - weight.shape[1])))
    out = _make_gather(N, V)(ids, w128, sc_flat)
    return out.reshape(B, L, D)


# speedup vs baseline: 1.7477x; 1.0985x over previous
import functools

import jax
import jax.numpy as jnp
from jax import lax
from jax.experimental import pallas as pl
from jax.experimental.pallas import tpu as pltpu
from jax.experimental.pallas import tpu_sc as plsc

D = 64
LANES = 16
NC = 2
NS = 16
NW = NC * NS

S = 200
SUB = 40


@functools.lru_cache(maxsize=None)
def _make_gather(B, L, V):
    N = B * L
    assert N % (NW * S) == 0 and S % L == 0
    per_w = N // NW
    n_chunks = per_w // S
    b_per_chunk = S // L
    mesh = plsc.VectorSubcoreMesh(core_axis_name="c", subcore_axis_name="s")

    @functools.partial(
        pl.kernel,
        mesh=mesh,
        compiler_params=pltpu.CompilerParams(use_tc_tiling_on_sc=True),
        out_type=jax.ShapeDtypeStruct((B, L, D), jnp.float32),
        scratch_types=[
            pltpu.VMEM((per_w,), jnp.int32),
            pltpu.VMEM((2, S, 2 * D), jnp.float32),
            pltpu.VMEM((S, D), jnp.float32),
            pltpu.VMEM((S,), jnp.float32),
            pltpu.VMEM((S,), jnp.float32),
            pltpu.SemaphoreType.DMA,
            pltpu.SemaphoreType.DMA,
        ],
    )
    def k(ids_hbm, w_hbm, sc_hbm, out_hbm,
          idx_v, rows_v, outb_v, scf0_v, scf1_v, sem0, sem1):
        wid = lax.axis_index("s") * NC + lax.axis_index("c")
        base = wid * per_w
        bbase = wid * (per_w // L)
        sems = (sem0, sem1)
        scfs = (scf0_v, scf1_v)

        pltpu.sync_copy(ids_hbm.at[pl.ds(base, per_w)], idx_v)

        def fire(c, b):
            for s in range(S // SUB):
                idx_s = idx_v.at[pl.ds(c * S + s * SUB, SUB)]
                pltpu.async_copy(sc_hbm.at[idx_s],
                                 scfs[b].at[pl.ds(s * SUB, SUB)], sems[b])
                pltpu.async_copy(w_hbm.at[idx_s],
                                 rows_v.at[b, pl.ds(s * SUB, SUB)], sems[b])

        def drain(c, b):
            for s in range(S // SUB):
                idx_s = idx_v.at[pl.ds(c * S + s * SUB, SUB)]
                pltpu.make_async_copy(
                    sc_hbm.at[idx_s],
                    scfs[b].at[pl.ds(s * SUB, SUB)], sems[b]).wait()
                pltpu.make_async_copy(
                    w_hbm.at[idx_s],
                    rows_v.at[b, pl.ds(s * SUB, SUB)], sems[b]).wait()

        def quant_rows(b, r0, sg, kks):
            nsg = -sg
            for kk in kks:
                splat = jnp.broadcast_to(sg[kk], (LANES,))
                nsplat = jnp.broadcast_to(nsg[kk], (LANES,))
                r = r0 + kk
                for j in range(D // LANES):
                    w = rows_v[b, r, pl.ds(j * LANES, LANES)]
                    outb_v[r, pl.ds(j * LANES, LANES)] = jnp.where(
                        w < 0, nsplat, splat)

        def compute(c, b):
            def group_body(g, carry2):
                sg = jnp.maximum(scfs[b][pl.ds(g * LANES, LANES)],
                                 jnp.float32(1e-8))
                quant_rows(b, g * LANES, sg, range(LANES))
                return carry2

            lax.fori_loop(0, S // LANES, group_body, 0)
            tail = S % LANES
            if tail:
                sg = jnp.maximum(scfs[b][pl.ds(S - LANES, LANES)],
                                 jnp.float32(1e-8))
                quant_rows(b, S - LANES, sg, range(LANES - tail, LANES))
            for i in range(b_per_chunk):
                pltpu.sync_copy(outb_v.at[pl.ds(i * L, L)],
                                out_hbm.at[bbase + c * b_per_chunk + i])

        fire(0, 0)

        def chunk_body(c2, carry):
            for b in range(2):
                c = c2 * 2 + b
                nxt = c + 1

                @pl.when(nxt < n_chunks)
                def _():
                    fire(nxt, 1 - b)

                drain(c, b)
                compute(c, b)
            return carry

        lax.fori_loop(0, n_chunks // 2, chunk_body, 0)

    return k


def kernel(input_ids, weight, scales):
    B, L = input_ids.shape
    V = weight.shape[0]
    N = B * L
    ids = input_ids.reshape(N).astype(jnp.int32)
    sc_flat = scales.reshape(-1)
    w128 = jnp.pad(weight, ((0, 0), (0, 2 * D - weight.shape[1])))
    return _make_gather(B, L, V)(ids, w128, sc_flat)
